# Initial kernel scaffold; baseline (speedup 1.0000x reference)
import jax
import jax.numpy as jnp
from jax.experimental import pallas as pl


def kernel(x, e_index, pos_enc, W, b):
    raise NotImplementedError("write your pallas kernel here")


# speedup vs baseline: 1.0249x; 1.0249x over previous
import jax
import jax.numpy as jnp
from jax.experimental import pallas as pl
from jax.experimental.pallas import tpu as pltpu

N_NODES_ = 10000
D_FEAT_ = 128
PE_DIM_ = 256
PE_EMBED_DIM_ = 512
BLOCK_M = 1000


def _fused_kernel(x_ref, pe_ref, w_ref, b_ref, out_ref):
    out_ref[:, :D_FEAT_] = x_ref[:]
    acc = jnp.dot(pe_ref[:], w_ref[:], preferred_element_type=jnp.float32)
    out_ref[:, D_FEAT_:] = acc + b_ref[:]


def kernel(x, e_index, pos_enc, W, b):
    del e_index
    n = x.shape[0]
    grid = (n // BLOCK_M,)
    out = pl.pallas_call(
        _fused_kernel,
        grid=grid,
        in_specs=[
            pl.BlockSpec((BLOCK_M, D_FEAT_), lambda i: (i, 0)),
            pl.BlockSpec((BLOCK_M, PE_DIM_), lambda i: (i, 0)),
            pl.BlockSpec((PE_DIM_, PE_EMBED_DIM_), lambda i: (0, 0)),
            pl.BlockSpec((PE_EMBED_DIM_,), lambda i: (0,)),
        ],
        out_specs=pl.BlockSpec((BLOCK_M, D_FEAT_ + PE_EMBED_DIM_), lambda i: (i, 0)),
        out_shape=jax.ShapeDtypeStruct((n, D_FEAT_ + PE_EMBED_DIM_), jnp.float32),
        compiler_params=pltpu.CompilerParams(
            dimension_semantics=("arbitrary",),
        ),
    )(x, pos_enc, W, b)
    return out


# speedup vs baseline: 1.1608x; 1.1326x over previous
import jax
import jax.numpy as jnp
from jax.experimental import pallas as pl
from jax.experimental.pallas import tpu as pltpu

N_NODES_ = 10000
D_FEAT_ = 128
PE_DIM_ = 256
PE_EMBED_DIM_ = 512
BLOCK_M = 2000


def _fused_kernel(x_ref, pe_ref, w_ref, b_ref, out_ref):
    out_ref[:, :D_FEAT_] = x_ref[:]
    acc = jnp.dot(pe_ref[:], w_ref[:], preferred_element_type=jnp.float32)
    out_ref[:, D_FEAT_:] = acc + b_ref[:]


def kernel(x, e_index, pos_enc, W, b):
    del e_index
    n = x.shape[0]
    grid = (n // BLOCK_M,)
    out = pl.pallas_call(
        _fused_kernel,
        grid=grid,
        in_specs=[
            pl.BlockSpec((BLOCK_M, D_FEAT_), lambda i: (i, 0)),
            pl.BlockSpec((BLOCK_M, PE_DIM_), lambda i: (i, 0)),
            pl.BlockSpec((PE_DIM_, PE_EMBED_DIM_), lambda i: (0, 0)),
            pl.BlockSpec((PE_EMBED_DIM_,), lambda i: (0,)),
        ],
        out_specs=pl.BlockSpec((BLOCK_M, D_FEAT_ + PE_EMBED_DIM_), lambda i: (i, 0)),
        out_shape=jax.ShapeDtypeStruct((n, D_FEAT_ + PE_EMBED_DIM_), jnp.float32),
        compiler_params=pltpu.CompilerParams(
            dimension_semantics=("arbitrary",),
        ),
    )(x, pos_enc, W, b)
    return out


# speedup vs baseline: 1.2371x; 1.0657x over previous
import jax
import jax.numpy as jnp
from jax.experimental import pallas as pl
from jax.experimental.pallas import tpu as pltpu

N_NODES_ = 10000
D_FEAT_ = 128
PE_DIM_ = 256
PE_EMBED_DIM_ = 512
BLOCK_M = 5000


def _fused_kernel(x_ref, pe_ref, w_ref, b_ref, out_ref):
    out_ref[:, :D_FEAT_] = x_ref[:]
    acc = jnp.dot(pe_ref[:], w_ref[:], preferred_element_type=jnp.float32)
    out_ref[:, D_FEAT_:] = acc + b_ref[:]


def kernel(x, e_index, pos_enc, W, b):
    del e_index
    n = x.shape[0]
    grid = (n // BLOCK_M,)
    out = pl.pallas_call(
        _fused_kernel,
        grid=grid,
        in_specs=[
            pl.BlockSpec((BLOCK_M, D_FEAT_), lambda i: (i, 0)),
            pl.BlockSpec((BLOCK_M, PE_DIM_), lambda i: (i, 0)),
            pl.BlockSpec((PE_DIM_, PE_EMBED_DIM_), lambda i: (0, 0)),
            pl.BlockSpec((PE_EMBED_DIM_,), lambda i: (0,)),
        ],
        out_specs=pl.BlockSpec((BLOCK_M, D_FEAT_ + PE_EMBED_DIM_), lambda i: (i, 0)),
        out_shape=jax.ShapeDtypeStruct((n, D_FEAT_ + PE_EMBED_DIM_), jnp.float32),
        compiler_params=pltpu.CompilerParams(
            dimension_semantics=("arbitrary",),
        ),
    )(x, pos_enc, W, b)
    return out


# speedup vs baseline: 1.2400x; 1.0024x over previous
import jax
import jax.numpy as jnp
from jax.experimental import pallas as pl
from jax.experimental.pallas import tpu as pltpu

N_NODES_ = 10000
D_FEAT_ = 128
PE_DIM_ = 256
PE_EMBED_DIM_ = 512
BLOCK_M = 5000


def _fused_kernel(x_ref, pe_ref, w_ref, b_ref, out_ref):
    out_ref[:, :D_FEAT_] = x_ref[:]
    acc = jnp.dot(pe_ref[:], w_ref[:], preferred_element_type=jnp.float32)
    out_ref[:, D_FEAT_:] = acc + b_ref[:]


def kernel(x, e_index, pos_enc, W, b):
    del e_index
    n = x.shape[0]
    grid = (n // BLOCK_M,)
    out = pl.pallas_call(
        _fused_kernel,
        grid=grid,
        in_specs=[
            pl.BlockSpec((BLOCK_M, D_FEAT_), lambda i: (i, 0)),
            pl.BlockSpec((BLOCK_M, PE_DIM_), lambda i: (i, 0)),
            pl.BlockSpec((PE_DIM_, PE_EMBED_DIM_), lambda i: (0, 0)),
            pl.BlockSpec((PE_EMBED_DIM_,), lambda i: (0,)),
        ],
        out_specs=pl.BlockSpec((BLOCK_M, D_FEAT_ + PE_EMBED_DIM_), lambda i: (i, 0)),
        out_shape=jax.ShapeDtypeStruct((n, D_FEAT_ + PE_EMBED_DIM_), jnp.float32),
        compiler_params=pltpu.CompilerParams(
            dimension_semantics=("parallel",),
        ),
    )(x, pos_enc, W, b)
    return out
